# baseline (speedup 1.0000x reference)
import functools

import jax
import jax.numpy as jnp
from jax import lax
from jax.experimental import pallas as pl
from jax.experimental.pallas import tpu as pltpu
from jax.experimental.pallas import tpu_sc as plsc

NUM_EMB = 100000
EMB_DIM = 32
BATCH = 16384

_L = 16
_CHUNK = 2048
_NCHUNK = BATCH // _CHUNK
_H = 50048


@functools.partial(
    pl.kernel,
    mesh=plsc.VectorSubcoreMesh(core_axis_name="c", subcore_axis_name="s"),
    out_type=jax.ShapeDtypeStruct((EMB_DIM, BATCH), jnp.float32),
    scratch_types=[
        pltpu.VMEM((1, NUM_EMB), jnp.float32),
        pltpu.VMEM((BATCH,), jnp.int32),
        pltpu.VMEM((2 * _CHUNK,), jnp.float32),
        pltpu.SemaphoreType.DMA,
        pltpu.SemaphoreType.DMA,
    ],
    compiler_params=pltpu.CompilerParams(
        needs_layout_passes=False,
        disable_bounds_checks=True,
        disable_semaphore_checks=True,
        skip_device_barrier=True,
    ),
)
def _gather_kernel(idx_hbm, tab_hbm, out_hbm, row_v, idx_v, out_v, sem_r, sem_w):
    j = lax.axis_index("s") * 2 + lax.axis_index("c")
    cp0 = pltpu.async_copy(
        tab_hbm.at[pl.ds(j, 1), pl.ds(0, _H)], row_v.at[:, pl.ds(0, _H)], sem_r
    )
    cp1 = pltpu.async_copy(
        tab_hbm.at[pl.ds(j, 1), pl.ds(_H, NUM_EMB - _H)],
        row_v.at[:, pl.ds(_H, NUM_EMB - _H)],
        sem_r,
    )
    idx_cp = pltpu.async_copy(idx_hbm, idx_v, sem_w)
    cp0.wait()
    cp1.wait()
    idx_cp.wait()

    def drain_one(c):
        pltpu.make_async_copy(
            out_v.at[pl.ds(0, _CHUNK)], out_hbm.at[j, pl.ds(0, _CHUNK)], sem_w
        ).wait()

    def chunk_body(c, carry):
        off = pl.multiple_of((c % 2) * _CHUNK, _CHUNK)
        base = pl.multiple_of(c * _CHUNK, _CHUNK)

        @pl.when(c >= 2)
        def _():
            drain_one(c)

        @plsc.parallel_loop(0, _CHUNK // _L, unroll=8)
        def gather_body(g):
            ivec = idx_v[pl.ds(base + g * _L, _L)]
            out_v[pl.ds(off + g * _L, _L)] = plsc.load_gather(
                row_v, [jnp.zeros((_L,), jnp.int32), ivec]
            )

        pltpu.async_copy(
            out_v.at[pl.ds(off, _CHUNK)], out_hbm.at[j, pl.ds(base, _CHUNK)], sem_w
        )
        return carry

    lax.fori_loop(0, _NCHUNK, chunk_body, 0)
    drain_one(0)
    drain_one(0)


def kernel(char_indices, table):
    out_t = _gather_kernel(char_indices.astype(jnp.int32), table.T)
    return out_t.T


# speedup vs baseline: 1.0020x; 1.0020x over previous
import functools

import jax
import jax.numpy as jnp
from jax import lax
from jax.experimental import pallas as pl
from jax.experimental.pallas import tpu as pltpu
from jax.experimental.pallas import tpu_sc as plsc

NUM_EMB = 100000
EMB_DIM = 32
BATCH = 16384

_L = 16
_CHUNK = 1024
_NCHUNK = BATCH // _CHUNK


@functools.partial(
    pl.kernel,
    mesh=plsc.VectorSubcoreMesh(core_axis_name="c", subcore_axis_name="s"),
    out_type=jax.ShapeDtypeStruct((EMB_DIM, BATCH), jnp.float32),
    scratch_types=[
        pltpu.VMEM((NUM_EMB,), jnp.float32),
        pltpu.VMEM((BATCH,), jnp.int32),
        pltpu.VMEM((2 * _CHUNK,), jnp.float32),
        pltpu.SemaphoreType.DMA,
        pltpu.SemaphoreType.DMA,
    ],
    compiler_params=pltpu.CompilerParams(
        needs_layout_passes=False,
        disable_bounds_checks=True,
        disable_semaphore_checks=True,
        skip_device_barrier=True,
    ),
)
def _gather_kernel(idx_hbm, tab_hbm, out_hbm, row_v, idx_v, out_v, sem_r, sem_w):
    j = lax.axis_index("s") * 2 + lax.axis_index("c")
    row_cp = pltpu.async_copy(tab_hbm.at[j], row_v, sem_r)
    idx_cp = pltpu.async_copy(idx_hbm, idx_v, sem_w)
    row_cp.wait()
    idx_cp.wait()

    def drain_one(c):
        pltpu.make_async_copy(
            out_v.at[pl.ds(0, _CHUNK)], out_hbm.at[j, pl.ds(0, _CHUNK)], sem_w
        ).wait()

    def chunk_body(c, carry):
        off = pl.multiple_of((c % 2) * _CHUNK, _CHUNK)
        base = pl.multiple_of(c * _CHUNK, _CHUNK)

        @pl.when(c >= 2)
        def _():
            drain_one(c)

        @plsc.parallel_loop(0, _CHUNK // _L, unroll=8)
        def gather_body(g):
            ivec = idx_v[pl.ds(base + g * _L, _L)]
            out_v[pl.ds(off + g * _L, _L)] = plsc.load_gather(row_v, [ivec])

        pltpu.async_copy(
            out_v.at[pl.ds(off, _CHUNK)], out_hbm.at[j, pl.ds(base, _CHUNK)], sem_w
        )
        return carry

    lax.fori_loop(0, _NCHUNK, chunk_body, 0)
    drain_one(0)
    drain_one(0)


def kernel(char_indices, table):
    out_t = _gather_kernel(char_indices.astype(jnp.int32), table.T)
    return out_t.T
